# baseline (speedup 1.0000x reference)
import functools

import jax
import jax.numpy as jnp
from jax import lax
from jax.experimental import pallas as pl
from jax.experimental.pallas import tpu as pltpu
from jax.experimental.pallas import tpu_sc as plsc

B = 16384
GMF_D = 32
MLP_D = 128
_W = 128
_WG = 128
_PACK_C = 4096


def _pack_gmf(eu_gmf_t, ei_gmf_t):
    f = jnp.arange(2 * GMF_D)[:, None]
    c = jnp.arange(MLP_D)[None, :]
    p = (c == f).astype(jnp.bfloat16)
    both_t = jnp.concatenate([eu_gmf_t.astype(jnp.bfloat16),
                              ei_gmf_t.astype(jnp.bfloat16)], axis=0)
    dn = (((0,), (0,)), ((), ()))
    return lax.dot_general(both_t, p, dn, preferred_element_type=jnp.float32)


@functools.cache
def _sc_mlp_gather_fn():
    mesh = plsc.VectorSubcoreMesh(core_axis_name="core",
                                  subcore_axis_name="subcore")

    @functools.partial(
        pl.kernel,
        out_type=(
            jax.ShapeDtypeStruct((B, MLP_D), jnp.float32),
            jax.ShapeDtypeStruct((B, MLP_D), jnp.float32),
        ),
        mesh=mesh,
        scratch_types=[pltpu.SemaphoreType.DMA] * 2,
    )
    def _sc_mlp(uidx_hbm, iidx_hbm, eu_mlp_hbm, ei_mlp_hbm, mu_hbm, mi_hbm,
                s0, s1):
        idx_spec = pl.BlockSpec((1, _W), lambda i: (0, i))
        row_spec = pl.BlockSpec((_W, MLP_D), lambda i: (i, 0))

        def body(u_v, i_v, mu_v, mi_v):
            c0 = pltpu.make_async_copy(eu_mlp_hbm.at[u_v.at[0]], mu_v, s0)
            c1 = pltpu.make_async_copy(ei_mlp_hbm.at[i_v.at[0]], mi_v, s1)
            c0.start(); c1.start()
            c0.wait(); c1.wait()

        pltpu.emit_pipeline(
            body,
            grid=(B // _W,),
            in_specs=[idx_spec, idx_spec],
            out_specs=[row_spec, row_spec],
            core_axis_name=("core", "subcore"),
            dimension_semantics=(pltpu.PARALLEL,),
        )(uidx_hbm, iidx_hbm, mu_hbm, mi_hbm)

    return _sc_mlp


@functools.cache
def _sc_gmf_gather_fn():
    mesh = plsc.VectorSubcoreMesh(core_axis_name="core",
                                  subcore_axis_name="subcore")

    @functools.partial(
        pl.kernel,
        out_type=jax.ShapeDtypeStruct((B, GMF_D), jnp.float32),
        mesh=mesh,
        scratch_types=[
            pltpu.VMEM((_WG, MLP_D), jnp.float32),
            pltpu.VMEM((_WG, MLP_D), jnp.float32),
            pltpu.SemaphoreType.DMA,
            pltpu.SemaphoreType.DMA,
        ],
    )
    def _sc_gmf(uidx_hbm, iidx_hbm, pack_hbm, g_hbm, wu_v, wi_v, s0, s1):
        idx_spec = pl.BlockSpec((1, _WG), lambda i: (0, i))

        def body(u_v, i_v, g_v):
            c0 = pltpu.make_async_copy(pack_hbm.at[u_v.at[0]], wu_v, s0)
            c1 = pltpu.make_async_copy(pack_hbm.at[i_v.at[0]], wi_v, s1)
            c0.start(); c1.start()
            c0.wait(); c1.wait()

            @pl.loop(0, _WG)
            def _(k):
                g_v[k, pl.ds(0, 16)] = (
                    wu_v[k, pl.ds(0, 16)] * wi_v[k, pl.ds(GMF_D, 16)])
                g_v[k, pl.ds(16, 16)] = (
                    wu_v[k, pl.ds(16, 16)] * wi_v[k, pl.ds(GMF_D + 16, 16)])

        pltpu.emit_pipeline(
            body,
            grid=(B // _WG,),
            in_specs=[idx_spec, idx_spec],
            out_specs=[pl.BlockSpec((_WG, GMF_D), lambda i: (i, 0))],
            core_axis_name=("core", "subcore"),
            dimension_semantics=(pltpu.PARALLEL,),
        )(uidx_hbm, iidx_hbm, g_hbm)

    return _sc_gmf


_BLK = 2048


def _tc_mlp_body(mu, mi, w0u, w0i, b0, w1, b1, w2, b2, wpm, out):
    h = jnp.dot(mu[...].astype(jnp.bfloat16), w0u[...].astype(jnp.bfloat16),
                preferred_element_type=jnp.float32)
    h = h + jnp.dot(mi[...].astype(jnp.bfloat16),
                    w0i[...].astype(jnp.bfloat16),
                    preferred_element_type=jnp.float32)
    h = jnp.maximum(h + b0[...], 0.0).astype(jnp.bfloat16)
    h = jnp.maximum(
        jnp.dot(h, w1[...].astype(jnp.bfloat16),
                preferred_element_type=jnp.float32) + b1[...],
        0.0).astype(jnp.bfloat16)
    h = jnp.maximum(
        jnp.dot(h, w2[...].astype(jnp.bfloat16),
                preferred_element_type=jnp.float32) + b2[...], 0.0)
    pm = jnp.dot(h, wpm[...], preferred_element_type=jnp.float32)
    out[...] = pm[:, 0]


def _tc_mlp(mu, mi, w0u, w0i, b0, w1, b1, w2, b2, wpm):
    n_blk = B // _BLK
    batch_spec = lambda d: pl.BlockSpec((_BLK, d), lambda i: (i, 0))
    full = lambda a: pl.BlockSpec(a.shape, lambda i: (0,) * a.ndim)
    return pl.pallas_call(
        _tc_mlp_body,
        grid=(n_blk,),
        in_specs=[
            batch_spec(MLP_D), batch_spec(MLP_D),
            full(w0u), full(w0i), full(b0), full(w1), full(b1),
            full(w2), full(b2), full(wpm),
        ],
        out_specs=pl.BlockSpec((_BLK,), lambda i: (i,)),
        out_shape=jax.ShapeDtypeStruct((B,), jnp.float32),
    )(mu, mi, w0u, w0i, b0, w1, b1, w2, b2, wpm)


def _tc_final_body(g, pm, wpg, bp, out):
    pg = jnp.dot(g[...], wpg[...], preferred_element_type=jnp.float32)
    out[...] = pg[:, 0] + pm[...] + bp[0, 0]


def _tc_final(g, pm, wpg, bp):
    full = lambda a: pl.BlockSpec(a.shape, lambda i: (0,) * a.ndim)
    return pl.pallas_call(
        _tc_final_body,
        grid=(1,),
        in_specs=[full(g), full(pm), full(wpg), full(bp)],
        out_specs=pl.BlockSpec((B,), lambda i: (i,)),
        out_shape=jax.ShapeDtypeStruct((B,), jnp.float32),
    )(g, pm, wpg, bp)


def kernel(x, eu_gmf, ei_gmf, eu_mlp, ei_mlp, W0, b0, W1, b1, W2, b2, Wp, bp):
    uidx = x[:, 0].reshape(1, B)
    iidx = x[:, 1].reshape(1, B)
    mu, mi = _sc_mlp_gather_fn()(uidx, iidx, eu_mlp, ei_mlp)
    gmf_pack = _pack_gmf(eu_gmf.T, ei_gmf.T)
    uidx2, iidx2, _ = lax.optimization_barrier((uidx, iidx, mu))
    g = _sc_gmf_gather_fn()(uidx2, iidx2, gmf_pack)
    pm = _tc_mlp(
        mu, mi,
        W0[:, :MLP_D].T, W0[:, MLP_D:].T, b0.reshape(1, -1),
        W1.T, b1.reshape(1, -1), W2.T, b2.reshape(1, -1),
        Wp[:, GMF_D:].T,
    )
    return _tc_final(g, pm, Wp[:, :GMF_D].T, bp.reshape(1, 1))


# speedup vs baseline: 1.0147x; 1.0147x over previous
import functools

import jax
import jax.numpy as jnp
from jax import lax
from jax.experimental import pallas as pl
from jax.experimental.pallas import tpu as pltpu
from jax.experimental.pallas import tpu_sc as plsc

B = 16384
GMF_D = 32
MLP_D = 128
_W = 128
_WG = 128
_PACK_C = 4096


def _pack_gmf(eu_gmf_t, ei_gmf_t):
    f = jnp.arange(2 * GMF_D)[:, None]
    c = jnp.arange(MLP_D)[None, :]
    p = (c == f).astype(jnp.bfloat16)
    both_t = jnp.concatenate([eu_gmf_t.astype(jnp.bfloat16),
                              ei_gmf_t.astype(jnp.bfloat16)], axis=0)
    dn = (((0,), (0,)), ((), ()))
    return lax.dot_general(both_t, p, dn, preferred_element_type=jnp.float32)


@functools.cache
def _sc_mlp_gather_fn():
    mesh = plsc.VectorSubcoreMesh(core_axis_name="core",
                                  subcore_axis_name="subcore")

    @functools.partial(
        pl.kernel,
        out_type=(
            jax.ShapeDtypeStruct((B, MLP_D), jnp.float32),
            jax.ShapeDtypeStruct((B, MLP_D), jnp.float32),
        ),
        mesh=mesh,
        scratch_types=[pltpu.SemaphoreType.DMA] * 2,
    )
    def _sc_mlp(uidx_hbm, iidx_hbm, eu_mlp_hbm, ei_mlp_hbm, mu_hbm, mi_hbm,
                s0, s1):
        idx_spec = pl.BlockSpec((1, _W), lambda i: (0, i))
        row_spec = pl.BlockSpec((_W, MLP_D), lambda i: (i, 0))

        def body(u_v, i_v, mu_v, mi_v):
            c0 = pltpu.make_async_copy(eu_mlp_hbm.at[u_v.at[0]], mu_v, s0)
            c1 = pltpu.make_async_copy(ei_mlp_hbm.at[i_v.at[0]], mi_v, s1)
            c0.start(); c1.start()
            c0.wait(); c1.wait()

        pltpu.emit_pipeline(
            body,
            grid=(B // _W,),
            in_specs=[idx_spec, idx_spec],
            out_specs=[row_spec, row_spec],
            core_axis_name=("core", "subcore"),
            dimension_semantics=(pltpu.PARALLEL,),
        )(uidx_hbm, iidx_hbm, mu_hbm, mi_hbm)

    return _sc_mlp


@functools.cache
def _sc_gmf_gather_fn():
    mesh = plsc.VectorSubcoreMesh(core_axis_name="core",
                                  subcore_axis_name="subcore")

    @functools.partial(
        pl.kernel,
        out_type=jax.ShapeDtypeStruct((B, GMF_D), jnp.float32),
        mesh=mesh,
        scratch_types=[
            pltpu.VMEM((_WG, MLP_D), jnp.float32),
            pltpu.VMEM((_WG, MLP_D), jnp.float32),
            pltpu.SemaphoreType.DMA,
            pltpu.SemaphoreType.DMA,
        ],
    )
    def _sc_gmf(uidx_hbm, iidx_hbm, pack_hbm, g_hbm, wu_v, wi_v, s0, s1):
        idx_spec = pl.BlockSpec((1, _WG), lambda i: (0, i))

        def body(u_v, i_v, g_v):
            c0 = pltpu.make_async_copy(pack_hbm.at[u_v.at[0]], wu_v, s0)
            c1 = pltpu.make_async_copy(pack_hbm.at[i_v.at[0]], wi_v, s1)
            c0.start(); c1.start()
            c0.wait(); c1.wait()

            @pl.loop(0, _WG)
            def _(k):
                g_v[k, pl.ds(0, 16)] = (
                    wu_v[k, pl.ds(0, 16)] * wi_v[k, pl.ds(GMF_D, 16)])
                g_v[k, pl.ds(16, 16)] = (
                    wu_v[k, pl.ds(16, 16)] * wi_v[k, pl.ds(GMF_D + 16, 16)])

        pltpu.emit_pipeline(
            body,
            grid=(B // _WG,),
            in_specs=[idx_spec, idx_spec],
            out_specs=[pl.BlockSpec((_WG, GMF_D), lambda i: (i, 0))],
            core_axis_name=("core", "subcore"),
            dimension_semantics=(pltpu.PARALLEL,),
        )(uidx_hbm, iidx_hbm, g_hbm)

    return _sc_gmf


_BLK = 2048


def _tc_mlp_body(mu, mi, w0u, w0i, b0, w1, b1, w2, b2, wpm, out):
    h = jnp.dot(mu[...].astype(jnp.bfloat16), w0u[...].astype(jnp.bfloat16),
                preferred_element_type=jnp.float32)
    h = h + jnp.dot(mi[...].astype(jnp.bfloat16),
                    w0i[...].astype(jnp.bfloat16),
                    preferred_element_type=jnp.float32)
    h = jnp.maximum(h + b0[...], 0.0).astype(jnp.bfloat16)
    h = jnp.maximum(
        jnp.dot(h, w1[...].astype(jnp.bfloat16),
                preferred_element_type=jnp.float32) + b1[...],
        0.0).astype(jnp.bfloat16)
    h = jnp.maximum(
        jnp.dot(h, w2[...].astype(jnp.bfloat16),
                preferred_element_type=jnp.float32) + b2[...], 0.0)
    pm = jnp.dot(h, wpm[...], preferred_element_type=jnp.float32)
    out[...] = pm[:, 0]


def _tc_mlp(mu, mi, w0u, w0i, b0, w1, b1, w2, b2, wpm):
    n_blk = B // _BLK
    batch_spec = lambda d: pl.BlockSpec((_BLK, d), lambda i: (i, 0))
    full = lambda a: pl.BlockSpec(a.shape, lambda i: (0,) * a.ndim)
    return pl.pallas_call(
        _tc_mlp_body,
        grid=(n_blk,),
        in_specs=[
            batch_spec(MLP_D), batch_spec(MLP_D),
            full(w0u), full(w0i), full(b0), full(w1), full(b1),
            full(w2), full(b2), full(wpm),
        ],
        out_specs=pl.BlockSpec((_BLK,), lambda i: (i,)),
        out_shape=jax.ShapeDtypeStruct((B,), jnp.float32),
    )(mu, mi, w0u, w0i, b0, w1, b1, w2, b2, wpm)


def _tc_final_body(g, pm, wpg, bp, out):
    pg = jnp.dot(g[...], wpg[...], preferred_element_type=jnp.float32)
    out[...] = pg[:, 0] + pm[...] + bp[0, 0]


def _tc_final(g, pm, wpg, bp):
    blk = 4096
    n_blk = B // blk
    full = lambda a: pl.BlockSpec(a.shape, lambda i: (0,) * a.ndim)
    return pl.pallas_call(
        _tc_final_body,
        grid=(n_blk,),
        in_specs=[
            pl.BlockSpec((blk, GMF_D), lambda i: (i, 0)),
            pl.BlockSpec((blk,), lambda i: (i,)),
            full(wpg), full(bp),
        ],
        out_specs=pl.BlockSpec((blk,), lambda i: (i,)),
        out_shape=jax.ShapeDtypeStruct((B,), jnp.float32),
    )(g, pm, wpg, bp)


def kernel(x, eu_gmf, ei_gmf, eu_mlp, ei_mlp, W0, b0, W1, b1, W2, b2, Wp, bp):
    uidx = x[:, 0].reshape(1, B)
    iidx = x[:, 1].reshape(1, B)
    mu, mi = _sc_mlp_gather_fn()(uidx, iidx, eu_mlp, ei_mlp)
    gmf_pack = _pack_gmf(eu_gmf.T, ei_gmf.T)
    uidx2, iidx2, _ = lax.optimization_barrier((uidx, iidx, mu))
    g = _sc_gmf_gather_fn()(uidx2, iidx2, gmf_pack)
    pm = _tc_mlp(
        mu, mi,
        W0[:, :MLP_D].T, W0[:, MLP_D:].T, b0.reshape(1, -1),
        W1.T, b1.reshape(1, -1), W2.T, b2.reshape(1, -1),
        Wp[:, GMF_D:].T,
    )
    return _tc_final(g, pm, Wp[:, :GMF_D].T, bp.reshape(1, 1))


# speedup vs baseline: 1.0386x; 1.0235x over previous
import functools

import jax
import jax.numpy as jnp
from jax import lax
from jax.experimental import pallas as pl
from jax.experimental.pallas import tpu as pltpu
from jax.experimental.pallas import tpu_sc as plsc

B = 16384
GMF_D = 32
MLP_D = 128
_W = 128
_WG = 128
_PACK_C = 4096


def _pack_gmf(eu_gmf_t, ei_gmf_t):
    f = jnp.arange(2 * GMF_D)[:, None]
    c = jnp.arange(MLP_D)[None, :]
    p = (c == f).astype(jnp.bfloat16)
    both_t = jnp.concatenate([eu_gmf_t.astype(jnp.bfloat16),
                              ei_gmf_t.astype(jnp.bfloat16)], axis=0)
    dn = (((0,), (0,)), ((), ()))
    return lax.dot_general(both_t, p, dn, preferred_element_type=jnp.float32)


@functools.cache
def _sc_mlp_gather_fn():
    mesh = plsc.VectorSubcoreMesh(core_axis_name="core",
                                  subcore_axis_name="subcore")

    @functools.partial(
        pl.kernel,
        out_type=(
            jax.ShapeDtypeStruct((B, MLP_D), jnp.float32),
            jax.ShapeDtypeStruct((B, MLP_D), jnp.float32),
        ),
        mesh=mesh,
        scratch_types=[pltpu.SemaphoreType.DMA] * 2,
    )
    def _sc_mlp(uidx_hbm, iidx_hbm, eu_mlp_hbm, ei_mlp_hbm, mu_hbm, mi_hbm,
                s0, s1):
        idx_spec = pl.BlockSpec((1, _W), lambda i: (0, i))
        row_spec = pl.BlockSpec((_W, MLP_D), lambda i: (i, 0))

        def body(u_v, i_v, mu_v, mi_v):
            c0 = pltpu.make_async_copy(eu_mlp_hbm.at[u_v.at[0]], mu_v, s0)
            c1 = pltpu.make_async_copy(ei_mlp_hbm.at[i_v.at[0]], mi_v, s1)
            c0.start(); c1.start()
            c0.wait(); c1.wait()

        pltpu.emit_pipeline(
            body,
            grid=(B // _W,),
            in_specs=[idx_spec, idx_spec],
            out_specs=[row_spec, row_spec],
            core_axis_name=("core", "subcore"),
            dimension_semantics=(pltpu.PARALLEL,),
        )(uidx_hbm, iidx_hbm, mu_hbm, mi_hbm)

    return _sc_mlp


@functools.cache
def _sc_gmf_gather_fn():
    mesh = plsc.VectorSubcoreMesh(core_axis_name="core",
                                  subcore_axis_name="subcore")

    @functools.partial(
        pl.kernel,
        out_type=jax.ShapeDtypeStruct((B, GMF_D), jnp.float32),
        mesh=mesh,
        scratch_types=[
            pltpu.VMEM((_WG, MLP_D), jnp.float32),
            pltpu.VMEM((_WG, MLP_D), jnp.float32),
            pltpu.SemaphoreType.DMA,
            pltpu.SemaphoreType.DMA,
        ],
    )
    def _sc_gmf(uidx_hbm, iidx_hbm, pack_hbm, g_hbm, wu_v, wi_v, s0, s1):
        idx_spec = pl.BlockSpec((1, _WG), lambda i: (0, i))

        def body(u_v, i_v, g_v):
            c0 = pltpu.make_async_copy(pack_hbm.at[u_v.at[0]], wu_v, s0)
            c1 = pltpu.make_async_copy(pack_hbm.at[i_v.at[0]], wi_v, s1)
            c0.start(); c1.start()
            c0.wait(); c1.wait()

            @pl.loop(0, _WG)
            def _(k):
                g_v[k, pl.ds(0, 16)] = (
                    wu_v[k, pl.ds(0, 16)] * wi_v[k, pl.ds(GMF_D, 16)])
                g_v[k, pl.ds(16, 16)] = (
                    wu_v[k, pl.ds(16, 16)] * wi_v[k, pl.ds(GMF_D + 16, 16)])

        pltpu.emit_pipeline(
            body,
            grid=(B // _WG,),
            in_specs=[idx_spec, idx_spec],
            out_specs=[pl.BlockSpec((_WG, GMF_D), lambda i: (i, 0))],
            core_axis_name=("core", "subcore"),
            dimension_semantics=(pltpu.PARALLEL,),
        )(uidx_hbm, iidx_hbm, g_hbm)

    return _sc_gmf


_BLK = 4096


def _tc_mlp_body(mu, mi, w0u, w0i, b0, w1, b1, w2, b2, wpm, out):
    h = jnp.dot(mu[...].astype(jnp.bfloat16), w0u[...].astype(jnp.bfloat16),
                preferred_element_type=jnp.float32)
    h = h + jnp.dot(mi[...].astype(jnp.bfloat16),
                    w0i[...].astype(jnp.bfloat16),
                    preferred_element_type=jnp.float32)
    h = jnp.maximum(h + b0[...], 0.0).astype(jnp.bfloat16)
    h = jnp.maximum(
        jnp.dot(h, w1[...].astype(jnp.bfloat16),
                preferred_element_type=jnp.float32) + b1[...],
        0.0).astype(jnp.bfloat16)
    h = jnp.maximum(
        jnp.dot(h, w2[...].astype(jnp.bfloat16),
                preferred_element_type=jnp.float32) + b2[...], 0.0)
    pm = jnp.dot(h, wpm[...], preferred_element_type=jnp.float32)
    out[...] = pm[:, 0]


def _tc_mlp(mu, mi, w0u, w0i, b0, w1, b1, w2, b2, wpm):
    n_blk = B // _BLK
    batch_spec = lambda d: pl.BlockSpec((_BLK, d), lambda i: (i, 0))
    full = lambda a: pl.BlockSpec(a.shape, lambda i: (0,) * a.ndim)
    return pl.pallas_call(
        _tc_mlp_body,
        grid=(n_blk,),
        in_specs=[
            batch_spec(MLP_D), batch_spec(MLP_D),
            full(w0u), full(w0i), full(b0), full(w1), full(b1),
            full(w2), full(b2), full(wpm),
        ],
        out_specs=pl.BlockSpec((_BLK,), lambda i: (i,)),
        out_shape=jax.ShapeDtypeStruct((B,), jnp.float32),
    )(mu, mi, w0u, w0i, b0, w1, b1, w2, b2, wpm)


def _tc_final_body(g, pm, wpg, bp, out):
    pg = jnp.dot(g[...], wpg[...], preferred_element_type=jnp.float32)
    out[...] = pg[:, 0] + pm[...] + bp[0, 0]


def _tc_final(g, pm, wpg, bp):
    blk = 4096
    n_blk = B // blk
    full = lambda a: pl.BlockSpec(a.shape, lambda i: (0,) * a.ndim)
    return pl.pallas_call(
        _tc_final_body,
        grid=(n_blk,),
        in_specs=[
            pl.BlockSpec((blk, GMF_D), lambda i: (i, 0)),
            pl.BlockSpec((blk,), lambda i: (i,)),
            full(wpg), full(bp),
        ],
        out_specs=pl.BlockSpec((blk,), lambda i: (i,)),
        out_shape=jax.ShapeDtypeStruct((B,), jnp.float32),
    )(g, pm, wpg, bp)


def kernel(x, eu_gmf, ei_gmf, eu_mlp, ei_mlp, W0, b0, W1, b1, W2, b2, Wp, bp):
    uidx = x[:, 0].reshape(1, B)
    iidx = x[:, 1].reshape(1, B)
    mu, mi = _sc_mlp_gather_fn()(uidx, iidx, eu_mlp, ei_mlp)
    gmf_pack = _pack_gmf(eu_gmf.T, ei_gmf.T)
    uidx2, iidx2, _ = lax.optimization_barrier((uidx, iidx, mu))
    g = _sc_gmf_gather_fn()(uidx2, iidx2, gmf_pack)
    pm = _tc_mlp(
        mu, mi,
        W0[:, :MLP_D].T, W0[:, MLP_D:].T, b0.reshape(1, -1),
        W1.T, b1.reshape(1, -1), W2.T, b2.reshape(1, -1),
        Wp[:, GMF_D:].T,
    )
    return _tc_final(g, pm, Wp[:, :GMF_D].T, bp.reshape(1, 1))
